# baseline (speedup 1.0000x reference)
import functools

import numpy as np

import jax
import jax.numpy as jnp
from jax.experimental import pallas as pl
from jax.experimental.pallas import tpu as pltpu


_OFFS = tuple((kd - 1, kh - 1, kw - 1)
              for kd in range(3) for kh in range(3) for kw in range(3))


def _rot(x, delta):
    size = x.shape[-1]
    if delta % size == 0:
        return x
    return pltpu.roll(x, (-delta) % size, x.ndim - 1)


def _build_patch(x, masks_ref, patch_ref, deltas):
    cin = x.shape[0]
    for k, delta in enumerate(deltas):
        shifted = _rot(x, delta) * masks_ref[k]
        patch_ref[k * cin:(k + 1) * cin, :] = shifted.astype(jnp.bfloat16)


def _dw_pair(a, b, wa_ref, wb_ref, masks_ref, deltas):
    acc_a = jnp.zeros(a.shape, jnp.float32)
    acc_b = jnp.zeros(b.shape, jnp.float32)
    for k, delta in enumerate(deltas):
        m = masks_ref[k]
        acc_a = acc_a + (_rot(a, delta) * m) * wa_ref[k]
        acc_b = acc_b + (_rot(b, delta) * m) * wb_ref[k]
    return acc_a, acc_b


def _moments(y, stats_ref):
    stats_ref[0, :, 0:1] = jnp.sum(y, axis=1, keepdims=True)
    stats_ref[0, :, 1:2] = jnp.sum(y * y, axis=1, keepdims=True)


def _s1_kernel(g_ref, w1_ref, masks_ref, g1_ref, stats_ref, patch_ref,
               *, deltas):
    g = g_ref[0]
    _build_patch(g, masks_ref, patch_ref, deltas)
    g1 = jnp.dot(w1_ref[...], patch_ref[...],
                 preferred_element_type=jnp.float32)
    g1_ref[0] = g1
    _moments(g1, stats_ref)


def _s2_kernel(x_ref, g1_ref, scale1_ref, shift1_ref, wxd_ref, wgd_ref,
               w2_ref, masks_ref, out_ref, stats_ref, patch_ref, *, deltas):
    x = x_ref[0]
    g1 = g1_ref[0] * scale1_ref[...] + shift1_ref[...]
    x_diff, g_diff = _dw_pair(x, g1, wxd_ref, wgd_ref, masks_ref, deltas)
    prod = x_diff * (g_diff * g_diff)
    _build_patch(prod, masks_ref, patch_ref, deltas)
    y = jnp.dot(w2_ref[...], patch_ref[...],
                preferred_element_type=jnp.float32)
    out_ref[0] = y
    _moments(y, stats_ref)


def _s3_kernel(y_ref, x_ref, scale_ref, shift_ref, o_ref):
    y = y_ref[0] * scale_ref[...] + shift_ref[...]
    o_ref[0] = jnp.maximum(y, 0.0) + x_ref[0]


def _stage1(gf, w1p, masks, deltas):
    N, G, S = gf.shape
    C = w1p.shape[0]
    fn = functools.partial(_s1_kernel, deltas=deltas)
    return pl.pallas_call(
        fn,
        out_shape=(jax.ShapeDtypeStruct((N, C, S), jnp.float32),
                   jax.ShapeDtypeStruct((N, C, 2), jnp.float32)),
        grid=(N,),
        in_specs=[
            pl.BlockSpec((1, G, S), lambda n: (n, 0, 0)),
            pl.BlockSpec((C, 27 * G), lambda n: (0, 0)),
            pl.BlockSpec((27, 1, S), lambda n: (0, 0, 0)),
        ],
        out_specs=(
            pl.BlockSpec((1, C, S), lambda n: (n, 0, 0)),
            pl.BlockSpec((1, C, 2), lambda n: (n, 0, 0)),
        ),
        scratch_shapes=[pltpu.VMEM((27 * G, S), jnp.bfloat16)],
        compiler_params=pltpu.CompilerParams(
            dimension_semantics=("parallel",)),
    )(gf, w1p, masks)


def _stage2(xf, g1_pre, scale1, shift1, wxd, wgd, w2p, masks, deltas):
    N, C, S = xf.shape
    fn = functools.partial(_s2_kernel, deltas=deltas)
    return pl.pallas_call(
        fn,
        out_shape=(jax.ShapeDtypeStruct((N, C, S), jnp.float32),
                   jax.ShapeDtypeStruct((N, C, 2), jnp.float32)),
        grid=(N,),
        in_specs=[
            pl.BlockSpec((1, C, S), lambda n: (n, 0, 0)),
            pl.BlockSpec((1, C, S), lambda n: (n, 0, 0)),
            pl.BlockSpec((C, 1), lambda n: (0, 0)),
            pl.BlockSpec((C, 1), lambda n: (0, 0)),
            pl.BlockSpec((27, C, 1), lambda n: (0, 0, 0)),
            pl.BlockSpec((27, C, 1), lambda n: (0, 0, 0)),
            pl.BlockSpec((C, 27 * C), lambda n: (0, 0)),
            pl.BlockSpec((27, 1, S), lambda n: (0, 0, 0)),
        ],
        out_specs=(
            pl.BlockSpec((1, C, S), lambda n: (n, 0, 0)),
            pl.BlockSpec((1, C, 2), lambda n: (n, 0, 0)),
        ),
        scratch_shapes=[pltpu.VMEM((27 * C, S), jnp.bfloat16)],
        compiler_params=pltpu.CompilerParams(
            dimension_semantics=("parallel",)),
    )(xf, g1_pre, scale1, shift1, wxd, wgd, w2p, masks)


def _stage3(y_pre, xf, scale2, shift2):
    N, C, S = xf.shape
    return pl.pallas_call(
        _s3_kernel,
        out_shape=jax.ShapeDtypeStruct((N, C, S), jnp.float32),
        grid=(N,),
        in_specs=[
            pl.BlockSpec((1, C, S), lambda n: (n, 0, 0)),
            pl.BlockSpec((1, C, S), lambda n: (n, 0, 0)),
            pl.BlockSpec((C, 1), lambda n: (0, 0)),
            pl.BlockSpec((C, 1), lambda n: (0, 0)),
        ],
        out_specs=pl.BlockSpec((1, C, S), lambda n: (n, 0, 0)),
        compiler_params=pltpu.CompilerParams(
            dimension_semantics=("parallel",)),
    )(y_pre, xf, scale2, shift2)


def _tap_tables(D, H, W):
    S = D * H * W
    deltas = tuple(od * H * W + oh * W + ow for od, oh, ow in _OFFS)
    d = np.arange(D)[:, None, None]
    h = np.arange(H)[None, :, None]
    w = np.arange(W)[None, None, :]
    masks = np.empty((27, 1, S), np.float32)
    for k, (od, oh, ow) in enumerate(_OFFS):
        valid = ((0 <= d + od) & (d + od < D) &
                 (0 <= h + oh) & (h + oh < H) &
                 (0 <= w + ow) & (w + ow < W))
        masks[k, 0, :] = valid.reshape(S).astype(np.float32)
    return deltas, jnp.asarray(masks)


def _bn_affine(stats, gamma, beta, count, eps):
    s = jnp.sum(stats, axis=0)
    mean = s[:, 0] / count
    var = jnp.maximum(s[:, 1] / count - mean * mean, 0.0)
    scale = gamma * jax.lax.rsqrt(var + eps)
    shift = beta - mean * scale
    return scale[:, None], shift[:, None]


def kernel(feature, guidance, conv1_w, conv_w, x_kernel_diff,
           guidance_kernel_diff, conv1_bn_gamma, conv1_bn_beta,
           bn_gamma, bn_beta, eps=1e-5):
    N, C, D, H, W = feature.shape
    G = guidance.shape[1]
    S = D * H * W
    deltas, masks = _tap_tables(D, H, W)

    xf = feature.reshape(N, C, S).astype(jnp.float32)
    gf = guidance.reshape(N, G, S).astype(jnp.float32)

    w1p = jnp.transpose(conv1_w, (2, 0, 1)).reshape(C, 27 * G)
    w1p = w1p.astype(jnp.bfloat16)
    w2p = jnp.transpose(conv_w, (2, 0, 1)).reshape(C, 27 * C)
    w2p = w2p.astype(jnp.bfloat16)
    wxd = x_kernel_diff[:, :, None]
    wgd = guidance_kernel_diff[:, :, None]

    g1_pre, stats1 = _stage1(gf, w1p, masks, deltas)
    scale1, shift1 = _bn_affine(stats1, conv1_bn_gamma, conv1_bn_beta,
                                N * S, eps)
    out_pre, stats2 = _stage2(xf, g1_pre, scale1, shift1,
                              wxd, wgd, w2p, masks, deltas)
    scale2, shift2 = _bn_affine(stats2, bn_gamma, bn_beta, N * S, eps)
    out = _stage3(out_pre, xf, scale2, shift2)
    return out.reshape(N, C, D, H, W)


# speedup vs baseline: 1.2899x; 1.2899x over previous
import functools

import numpy as np

import jax
import jax.numpy as jnp
from jax.experimental import pallas as pl
from jax.experimental.pallas import tpu as pltpu

_F32 = jnp.float32
_BF16 = jnp.bfloat16


def _rot(x, delta):
    size = x.shape[-1]
    if delta % size == 0:
        return x
    return pltpu.roll(x, (-delta) % size, x.ndim - 1)


def _build_r(y, mhw_ref, r_ref, hw_deltas):
    rows = y.shape[0]
    for j, delta in enumerate(hw_deltas):
        r = _rot(y, delta)
        if j != 4:
            r = r * mhw_ref[j]
        r_ref[j * rows:(j + 1) * rows, :] = r.astype(_BF16)


def _od_combine(z, md_ref, rows, hw_span):
    lo = _rot(z[0:rows], -hw_span) * md_ref[0]
    hi = _rot(z[2 * rows:3 * rows], hw_span) * md_ref[1]
    return z[rows:2 * rows] + lo + hi


def _moments(y, stats_ref):
    stats_ref[0, :, 0:1] = jnp.sum(y, axis=1, keepdims=True)
    stats_ref[0, :, 1:2] = jnp.sum(y * y, axis=1, keepdims=True)


def _s1_kernel(g_ref, w1_ref, mhw_ref, md_ref, g1_ref, stats_ref, r_ref,
               *, hw_deltas, hw_span):
    g = g_ref[0]
    _build_r(g, mhw_ref, r_ref, hw_deltas)
    z = jnp.dot(w1_ref[...], r_ref[...], preferred_element_type=_F32)
    g1 = _od_combine(z, md_ref, w1_ref.shape[0] // 3, hw_span)
    _moments(g1, stats_ref)
    g1_ref[0] = g1.astype(_BF16)


def _s2_kernel(x_ref, g1_ref, scale1_ref, shift1_ref, wdw_ref, w2_ref,
               mhw_ref, md_ref, out_ref, stats_ref, rxg_ref, rp_ref,
               *, hw_deltas, hw_span):
    x = x_ref[0]
    c = x.shape[0]
    g1 = g1_ref[0].astype(_F32) * scale1_ref[...] + shift1_ref[...]
    xg = jnp.concatenate([x, g1], axis=0)
    _build_r(xg, mhw_ref, rxg_ref, hw_deltas)
    zdw = jnp.dot(wdw_ref[...], rxg_ref[...], preferred_element_type=_F32)
    dw = _od_combine(zdw, md_ref, 2 * c, hw_span)
    g_diff = dw[c:]
    prod = dw[:c] * (g_diff * g_diff)
    _build_r(prod, mhw_ref, rp_ref, hw_deltas)
    z2 = jnp.dot(w2_ref[...], rp_ref[...], preferred_element_type=_F32)
    y = _od_combine(z2, md_ref, c, hw_span)
    _moments(y, stats_ref)
    out_ref[0] = y.astype(_BF16)


def _s3_kernel(y_ref, x_ref, scale_ref, shift_ref, o_ref):
    y = y_ref[0].astype(_F32) * scale_ref[...] + shift_ref[...]
    o_ref[0] = jnp.maximum(y, 0.0) + x_ref[0]


def _stage1(gf, w1od, mhw, md, hw_deltas, hw_span):
    N, G, S = gf.shape
    C = w1od.shape[0] // 3
    fn = functools.partial(_s1_kernel, hw_deltas=hw_deltas, hw_span=hw_span)
    return pl.pallas_call(
        fn,
        out_shape=(jax.ShapeDtypeStruct((N, C, S), _BF16),
                   jax.ShapeDtypeStruct((N, C, 2), _F32)),
        grid=(N,),
        in_specs=[
            pl.BlockSpec((1, G, S), lambda n: (n, 0, 0)),
            pl.BlockSpec((3 * C, 9 * G), lambda n: (0, 0)),
            pl.BlockSpec((9, 1, S), lambda n: (0, 0, 0)),
            pl.BlockSpec((2, 1, S), lambda n: (0, 0, 0)),
        ],
        out_specs=(
            pl.BlockSpec((1, C, S), lambda n: (n, 0, 0)),
            pl.BlockSpec((1, C, 2), lambda n: (n, 0, 0)),
        ),
        scratch_shapes=[pltpu.VMEM((9 * G, S), _BF16)],
        compiler_params=pltpu.CompilerParams(
            dimension_semantics=("parallel",)),
    )(gf, w1od, mhw, md)


def _stage2(xf, g1_pre, scale1, shift1, wdw, w2od, mhw, md, hw_deltas,
            hw_span):
    N, C, S = xf.shape
    fn = functools.partial(_s2_kernel, hw_deltas=hw_deltas, hw_span=hw_span)
    return pl.pallas_call(
        fn,
        out_shape=(jax.ShapeDtypeStruct((N, C, S), _BF16),
                   jax.ShapeDtypeStruct((N, C, 2), _F32)),
        grid=(N,),
        in_specs=[
            pl.BlockSpec((1, C, S), lambda n: (n, 0, 0)),
            pl.BlockSpec((1, C, S), lambda n: (n, 0, 0)),
            pl.BlockSpec((C, 1), lambda n: (0, 0)),
            pl.BlockSpec((C, 1), lambda n: (0, 0)),
            pl.BlockSpec((3 * 2 * C, 9 * 2 * C), lambda n: (0, 0)),
            pl.BlockSpec((3 * C, 9 * C), lambda n: (0, 0)),
            pl.BlockSpec((9, 1, S), lambda n: (0, 0, 0)),
            pl.BlockSpec((2, 1, S), lambda n: (0, 0, 0)),
        ],
        out_specs=(
            pl.BlockSpec((1, C, S), lambda n: (n, 0, 0)),
            pl.BlockSpec((1, C, 2), lambda n: (n, 0, 0)),
        ),
        scratch_shapes=[pltpu.VMEM((9 * 2 * C, S), _BF16),
                        pltpu.VMEM((9 * C, S), _BF16)],
        compiler_params=pltpu.CompilerParams(
            dimension_semantics=("parallel",)),
    )(xf, g1_pre, scale1, shift1, wdw, w2od, mhw, md)


def _stage3(y_pre, xf, scale2, shift2):
    N, C, S = xf.shape
    return pl.pallas_call(
        _s3_kernel,
        out_shape=jax.ShapeDtypeStruct((N, C, S), _F32),
        grid=(N,),
        in_specs=[
            pl.BlockSpec((1, C, S), lambda n: (n, 0, 0)),
            pl.BlockSpec((1, C, S), lambda n: (n, 0, 0)),
            pl.BlockSpec((C, 1), lambda n: (0, 0)),
            pl.BlockSpec((C, 1), lambda n: (0, 0)),
        ],
        out_specs=pl.BlockSpec((1, C, S), lambda n: (n, 0, 0)),
        compiler_params=pltpu.CompilerParams(
            dimension_semantics=("parallel",)),
    )(y_pre, xf, scale2, shift2)


def _tap_tables(D, H, W):
    S = D * H * W
    hw_deltas = tuple(oh * W + ow for oh in (-1, 0, 1) for ow in (-1, 0, 1))
    h = np.arange(H)[:, None]
    w = np.arange(W)[None, :]
    mhw = np.empty((9, 1, S), np.float32)
    for j, (oh, ow) in enumerate((a, b) for a in (-1, 0, 1)
                                 for b in (-1, 0, 1)):
        valid = ((0 <= h + oh) & (h + oh < H) &
                 (0 <= w + ow) & (w + ow < W))
        mhw[j, 0, :] = np.tile(valid.reshape(H * W), D).astype(np.float32)
    d = np.repeat(np.arange(D), H * W)
    md = np.stack([(d - 1 >= 0).astype(np.float32),
                   (d + 1 < D).astype(np.float32)])[:, None, :]
    return hw_deltas, jnp.asarray(mhw), jnp.asarray(md)


def _pack_od(w, cout):
    k, cin, _ = w.shape
    w = w.reshape(3, 9, cin, cout)
    return jnp.transpose(w, (0, 3, 1, 2)).reshape(3 * cout, 9 * cin) \
        .astype(_BF16)


def _pack_dw(wx, wg):
    c2 = wx.shape[1] * 2
    wd = jnp.concatenate([wx, wg], axis=1).reshape(3, 9, c2)
    eye = jnp.eye(c2, dtype=_F32)
    blk = wd[:, :, None, :] * eye[None, None]
    return jnp.transpose(blk, (0, 2, 1, 3)).reshape(3 * c2, 9 * c2) \
        .astype(_BF16)


def _bn_affine(stats, gamma, beta, count, eps):
    s = jnp.sum(stats, axis=0)
    mean = s[:, 0] / count
    var = jnp.maximum(s[:, 1] / count - mean * mean, 0.0)
    scale = gamma * jax.lax.rsqrt(var + eps)
    shift = beta - mean * scale
    return scale[:, None], shift[:, None]


def kernel(feature, guidance, conv1_w, conv_w, x_kernel_diff,
           guidance_kernel_diff, conv1_bn_gamma, conv1_bn_beta,
           bn_gamma, bn_beta, eps=1e-5):
    N, C, D, H, W = feature.shape
    G = guidance.shape[1]
    S = D * H * W
    hw_span = H * W
    hw_deltas, mhw, md = _tap_tables(D, H, W)

    xf = feature.reshape(N, C, S).astype(_F32)
    gf = guidance.reshape(N, G, S).astype(_F32)

    w1od = _pack_od(conv1_w, C)
    w2od = _pack_od(conv_w, C)
    wdw = _pack_dw(x_kernel_diff, guidance_kernel_diff)

    g1_pre, stats1 = _stage1(gf, w1od, mhw, md, hw_deltas, hw_span)
    scale1, shift1 = _bn_affine(stats1, conv1_bn_gamma, conv1_bn_beta,
                                N * S, eps)
    out_pre, stats2 = _stage2(xf, g1_pre, scale1, shift1, wdw, w2od,
                              mhw, md, hw_deltas, hw_span)
    scale2, shift2 = _bn_affine(stats2, bn_gamma, bn_beta, N * S, eps)
    out = _stage3(out_pre, xf, scale2, shift2)
    return out.reshape(N, C, D, H, W)


# speedup vs baseline: 1.7036x; 1.3207x over previous
import functools

import numpy as np

import jax
import jax.numpy as jnp
from jax.experimental import pallas as pl
from jax.experimental.pallas import tpu as pltpu

_F32 = jnp.float32
_BF16 = jnp.bfloat16


def _rot(x, delta):
    size = x.shape[-1]
    if delta % size == 0:
        return x
    return pltpu.roll(x, (-delta) % size, x.ndim - 1)


def _rot_bf(x, delta):
    size = x.shape[-1]
    d = delta % size
    if d == 0:
        return x
    return jnp.concatenate([x[..., d:], x[..., :d]], axis=-1)


def _build_r(y, mhw_ref, r_ref, hw_deltas, row0=0):
    rows = y.shape[0]
    stride = r_ref.shape[0] // len(hw_deltas)
    for j, delta in enumerate(hw_deltas):
        r = _rot_bf(y, delta)
        if j != 4:
            r = r * mhw_ref[j]
        base = j * stride + row0
        r_ref[base:base + rows, :] = r


def _od_combine(z, md_ref, rows, hw_span):
    lo = _rot(z[0:rows], -hw_span) * md_ref[0]
    hi = _rot(z[2 * rows:3 * rows], hw_span) * md_ref[1]
    return z[rows:2 * rows] + lo + hi


def _moments(y, stats_ref):
    stats_ref[0, :, 0:1] = jnp.sum(y, axis=1, keepdims=True)
    stats_ref[0, :, 1:2] = jnp.sum(y * y, axis=1, keepdims=True)


def _s1_kernel(g_ref, w1_ref, mhw_ref, md_ref, g1_ref, stats_ref, r_ref,
               *, hw_deltas, hw_span):
    g = g_ref[0].astype(_BF16)
    _build_r(g, mhw_ref, r_ref, hw_deltas)
    z = jnp.dot(w1_ref[...], r_ref[...], preferred_element_type=_F32)
    g1 = _od_combine(z, md_ref, w1_ref.shape[0] // 3, hw_span)
    _moments(g1, stats_ref)
    g1_ref[0] = g1.astype(_BF16)


def _s2_kernel(x_ref, g1_ref, scale1_ref, shift1_ref, wdw_ref, w2_ref,
               mhw_ref, md_ref, out_ref, stats_ref, rxg_ref, rp_ref,
               *, hw_deltas, hw_span):
    x = x_ref[0].astype(_BF16)
    c = x.shape[0]
    g1 = (g1_ref[0] * scale1_ref[...].astype(_BF16)
          + shift1_ref[...].astype(_BF16))
    _build_r(x, mhw_ref, rxg_ref, hw_deltas)
    _build_r(g1, mhw_ref, rxg_ref, hw_deltas, row0=c)
    zdw = jnp.dot(wdw_ref[...], rxg_ref[...], preferred_element_type=_F32)
    dw = _od_combine(zdw, md_ref, 2 * c, hw_span)
    g_diff = dw[c:]
    prod = (dw[:c] * (g_diff * g_diff)).astype(_BF16)
    _build_r(prod, mhw_ref, rp_ref, hw_deltas)
    z2 = jnp.dot(w2_ref[...], rp_ref[...], preferred_element_type=_F32)
    y = _od_combine(z2, md_ref, c, hw_span)
    _moments(y, stats_ref)
    out_ref[0] = y.astype(_BF16)


def _s3_kernel(y_ref, x_ref, scale_ref, shift_ref, o_ref):
    y = y_ref[0].astype(_F32) * scale_ref[...] + shift_ref[...]
    o_ref[0] = jnp.maximum(y, 0.0) + x_ref[0]


def _stage1(gf, w1od, mhw, md, hw_deltas, hw_span):
    N, G, S = gf.shape
    C = w1od.shape[0] // 3
    fn = functools.partial(_s1_kernel, hw_deltas=hw_deltas, hw_span=hw_span)
    return pl.pallas_call(
        fn,
        out_shape=(jax.ShapeDtypeStruct((N, C, S), _BF16),
                   jax.ShapeDtypeStruct((N, C, 2), _F32)),
        grid=(N,),
        in_specs=[
            pl.BlockSpec((1, G, S), lambda n: (n, 0, 0)),
            pl.BlockSpec((3 * C, 9 * G), lambda n: (0, 0)),
            pl.BlockSpec((9, 1, S), lambda n: (0, 0, 0)),
            pl.BlockSpec((2, 1, S), lambda n: (0, 0, 0)),
        ],
        out_specs=(
            pl.BlockSpec((1, C, S), lambda n: (n, 0, 0)),
            pl.BlockSpec((1, C, 2), lambda n: (n, 0, 0)),
        ),
        scratch_shapes=[pltpu.VMEM((9 * G, S), _BF16)],
        compiler_params=pltpu.CompilerParams(
            dimension_semantics=("parallel",)),
    )(gf, w1od, mhw, md)


def _stage2(xf, g1_pre, scale1, shift1, wdw, w2od, mhw, md, hw_deltas,
            hw_span):
    N, C, S = xf.shape
    fn = functools.partial(_s2_kernel, hw_deltas=hw_deltas, hw_span=hw_span)
    return pl.pallas_call(
        fn,
        out_shape=(jax.ShapeDtypeStruct((N, C, S), _BF16),
                   jax.ShapeDtypeStruct((N, C, 2), _F32)),
        grid=(N,),
        in_specs=[
            pl.BlockSpec((1, C, S), lambda n: (n, 0, 0)),
            pl.BlockSpec((1, C, S), lambda n: (n, 0, 0)),
            pl.BlockSpec((C, 1), lambda n: (0, 0)),
            pl.BlockSpec((C, 1), lambda n: (0, 0)),
            pl.BlockSpec((3 * 2 * C, 9 * 2 * C), lambda n: (0, 0)),
            pl.BlockSpec((3 * C, 9 * C), lambda n: (0, 0)),
            pl.BlockSpec((9, 1, S), lambda n: (0, 0, 0)),
            pl.BlockSpec((2, 1, S), lambda n: (0, 0, 0)),
        ],
        out_specs=(
            pl.BlockSpec((1, C, S), lambda n: (n, 0, 0)),
            pl.BlockSpec((1, C, 2), lambda n: (n, 0, 0)),
        ),
        scratch_shapes=[pltpu.VMEM((9 * 2 * C, S), _BF16),
                        pltpu.VMEM((9 * C, S), _BF16)],
        compiler_params=pltpu.CompilerParams(
            dimension_semantics=("parallel",)),
    )(xf, g1_pre, scale1, shift1, wdw, w2od, mhw, md)


def _stage3(y_pre, xf, scale2, shift2):
    N, C, S = xf.shape
    return pl.pallas_call(
        _s3_kernel,
        out_shape=jax.ShapeDtypeStruct((N, C, S), _F32),
        grid=(N,),
        in_specs=[
            pl.BlockSpec((1, C, S), lambda n: (n, 0, 0)),
            pl.BlockSpec((1, C, S), lambda n: (n, 0, 0)),
            pl.BlockSpec((C, 1), lambda n: (0, 0)),
            pl.BlockSpec((C, 1), lambda n: (0, 0)),
        ],
        out_specs=pl.BlockSpec((1, C, S), lambda n: (n, 0, 0)),
        compiler_params=pltpu.CompilerParams(
            dimension_semantics=("parallel",)),
    )(y_pre, xf, scale2, shift2)


def _tap_tables(D, H, W):
    S = D * H * W
    hw_deltas = tuple(oh * W + ow for oh in (-1, 0, 1) for ow in (-1, 0, 1))
    h = np.arange(H)[:, None]
    w = np.arange(W)[None, :]
    mhw = np.empty((9, 1, S), np.float32)
    for j, (oh, ow) in enumerate((a, b) for a in (-1, 0, 1)
                                 for b in (-1, 0, 1)):
        valid = ((0 <= h + oh) & (h + oh < H) &
                 (0 <= w + ow) & (w + ow < W))
        mhw[j, 0, :] = np.tile(valid.reshape(H * W), D).astype(np.float32)
    d = np.repeat(np.arange(D), H * W)
    md = np.stack([(d - 1 >= 0).astype(np.float32),
                   (d + 1 < D).astype(np.float32)])[:, None, :]
    return hw_deltas, jnp.asarray(mhw, _BF16), jnp.asarray(md)


def _pack_od(w, cout):
    k, cin, _ = w.shape
    w = w.reshape(3, 9, cin, cout)
    return jnp.transpose(w, (0, 3, 1, 2)).reshape(3 * cout, 9 * cin) \
        .astype(_BF16)


def _pack_dw(wx, wg):
    c2 = wx.shape[1] * 2
    wd = jnp.concatenate([wx, wg], axis=1).reshape(3, 9, c2)
    eye = jnp.eye(c2, dtype=_F32)
    blk = wd[:, :, None, :] * eye[None, None]
    return jnp.transpose(blk, (0, 2, 1, 3)).reshape(3 * c2, 9 * c2) \
        .astype(_BF16)


def _bn_affine(stats, gamma, beta, count, eps):
    s = jnp.sum(stats, axis=0)
    mean = s[:, 0] / count
    var = jnp.maximum(s[:, 1] / count - mean * mean, 0.0)
    scale = gamma * jax.lax.rsqrt(var + eps)
    shift = beta - mean * scale
    return scale[:, None], shift[:, None]


def kernel(feature, guidance, conv1_w, conv_w, x_kernel_diff,
           guidance_kernel_diff, conv1_bn_gamma, conv1_bn_beta,
           bn_gamma, bn_beta, eps=1e-5):
    N, C, D, H, W = feature.shape
    G = guidance.shape[1]
    S = D * H * W
    hw_span = H * W
    hw_deltas, mhw, md = _tap_tables(D, H, W)

    xf = feature.reshape(N, C, S).astype(_F32)
    gf = guidance.reshape(N, G, S).astype(_F32)

    w1od = _pack_od(conv1_w, C)
    w2od = _pack_od(conv_w, C)
    wdw = _pack_dw(x_kernel_diff, guidance_kernel_diff)

    g1_pre, stats1 = _stage1(gf, w1od, mhw, md, hw_deltas, hw_span)
    scale1, shift1 = _bn_affine(stats1, conv1_bn_gamma, conv1_bn_beta,
                                N * S, eps)
    out_pre, stats2 = _stage2(xf, g1_pre, scale1, shift1, wdw, w2od,
                              mhw, md, hw_deltas, hw_span)
    scale2, shift2 = _bn_affine(stats2, bn_gamma, bn_beta, N * S, eps)
    out = _stage3(out_pre, xf, scale2, shift2)
    return out.reshape(N, C, D, H, W)


# speedup vs baseline: 1.7475x; 1.0258x over previous
import functools

import numpy as np

import jax
import jax.numpy as jnp
from jax.experimental import pallas as pl
from jax.experimental.pallas import tpu as pltpu

_F32 = jnp.float32
_BF16 = jnp.bfloat16


def _rot(x, delta):
    size = x.shape[-1]
    if delta % size == 0:
        return x
    return pltpu.roll(x, (-delta) % size, x.ndim - 1)


def _rot_bf(x, delta):
    size = x.shape[-1]
    d = delta % size
    if d == 0:
        return x
    return jnp.concatenate([x[..., d:], x[..., :d]], axis=-1)


def _build_r(y, mhw_ref, r_ref, hw_deltas, row0=0):
    rows = y.shape[0]
    stride = r_ref.shape[0] // len(hw_deltas)
    for j, delta in enumerate(hw_deltas):
        r = _rot_bf(y, delta)
        if j != 4:
            r = r * mhw_ref[j]
        base = j * stride + row0
        r_ref[base:base + rows, :] = r


def _od_combine(z, md_ref, rows, hw_span):
    lo = _rot(z[0:rows], -hw_span) * md_ref[0]
    hi = _rot(z[2 * rows:3 * rows], hw_span) * md_ref[1]
    return z[rows:2 * rows] + lo + hi


def _moments(y, stats_ref, b):
    stats_ref[b, :, 0:1] = jnp.sum(y, axis=1, keepdims=True)
    stats_ref[b, :, 1:2] = jnp.sum(y * y, axis=1, keepdims=True)


def _affine(stats_ref, gam_ref, bet_ref, count, eps):
    s = jnp.sum(stats_ref[...], axis=0)
    mean = s[:, 0:1] / count
    var = jnp.maximum(s[:, 1:2] / count - mean * mean, 0.0)
    scale = gam_ref[...] * jax.lax.rsqrt(var + eps)
    shift = bet_ref[...] - mean * scale
    return scale, shift


def _s1_kernel(g_ref, w1_ref, mhw_ref, md_ref, g1_ref, stats_ref, r_ref,
               *, hw_deltas, hw_span, nb):
    c = w1_ref.shape[0] // 3
    for b in range(nb):
        g = g_ref[b].astype(_BF16)
        _build_r(g, mhw_ref, r_ref, hw_deltas)
        z = jnp.dot(w1_ref[...], r_ref[...], preferred_element_type=_F32)
        g1 = _od_combine(z, md_ref, c, hw_span)
        _moments(g1, stats_ref, b)
        g1_ref[b] = g1.astype(_BF16)


def _s2_kernel(x_ref, g1_ref, stats1_ref, gam1_ref, bet1_ref, wdw_ref,
               w2_ref, mhw_ref, md_ref, out_ref, stats_ref, rxg_ref, rp_ref,
               *, hw_deltas, hw_span, nb, count, eps):
    scale, shift = _affine(stats1_ref, gam1_ref, bet1_ref, count, eps)
    scale = scale.astype(_BF16)
    shift = shift.astype(_BF16)
    for b in range(nb):
        x = x_ref[b].astype(_BF16)
        c = x.shape[0]
        g1 = g1_ref[b] * scale + shift
        _build_r(x, mhw_ref, rxg_ref, hw_deltas)
        _build_r(g1, mhw_ref, rxg_ref, hw_deltas, row0=c)
        zdw = jnp.dot(wdw_ref[...], rxg_ref[...],
                      preferred_element_type=_F32)
        dw = _od_combine(zdw, md_ref, 2 * c, hw_span)
        g_diff = dw[c:]
        prod = (dw[:c] * (g_diff * g_diff)).astype(_BF16)
        _build_r(prod, mhw_ref, rp_ref, hw_deltas)
        z2 = jnp.dot(w2_ref[...], rp_ref[...], preferred_element_type=_F32)
        y = _od_combine(z2, md_ref, c, hw_span)
        _moments(y, stats_ref, b)
        out_ref[b] = y.astype(_BF16)


def _s3_kernel(y_ref, x_ref, stats2_ref, gam_ref, bet_ref, o_ref,
               *, nb, count, eps):
    scale, shift = _affine(stats2_ref, gam_ref, bet_ref, count, eps)
    for b in range(nb):
        y = y_ref[b].astype(_F32) * scale + shift
        o_ref[b] = jnp.maximum(y, 0.0) + x_ref[b]


def _pick_nb(n, cap):
    for b in range(cap, 0, -1):
        if n % b == 0:
            return b
    return 1


def _stage1(gf, w1od, mhw, md, hw_deltas, hw_span):
    N, G, S = gf.shape
    C = w1od.shape[0] // 3
    nb = _pick_nb(N, 4)
    fn = functools.partial(_s1_kernel, hw_deltas=hw_deltas, hw_span=hw_span,
                           nb=nb)
    return pl.pallas_call(
        fn,
        out_shape=(jax.ShapeDtypeStruct((N, C, S), _BF16),
                   jax.ShapeDtypeStruct((N, C, 2), _F32)),
        grid=(N // nb,),
        in_specs=[
            pl.BlockSpec((nb, G, S), lambda n: (n, 0, 0)),
            pl.BlockSpec((3 * C, 9 * G), lambda n: (0, 0)),
            pl.BlockSpec((9, 1, S), lambda n: (0, 0, 0)),
            pl.BlockSpec((2, 1, S), lambda n: (0, 0, 0)),
        ],
        out_specs=(
            pl.BlockSpec((nb, C, S), lambda n: (n, 0, 0)),
            pl.BlockSpec((nb, C, 2), lambda n: (n, 0, 0)),
        ),
        scratch_shapes=[pltpu.VMEM((9 * G, S), _BF16)],
        compiler_params=pltpu.CompilerParams(
            dimension_semantics=("parallel",)),
    )(gf, w1od, mhw, md)


def _stage2(xf, g1_pre, stats1, gam1, bet1, wdw, w2od, mhw, md, hw_deltas,
            hw_span, eps):
    N, C, S = xf.shape
    nb = _pick_nb(N, 2)
    fn = functools.partial(_s2_kernel, hw_deltas=hw_deltas, hw_span=hw_span,
                           nb=nb, count=float(N * S), eps=eps)
    return pl.pallas_call(
        fn,
        out_shape=(jax.ShapeDtypeStruct((N, C, S), _BF16),
                   jax.ShapeDtypeStruct((N, C, 2), _F32)),
        grid=(N // nb,),
        in_specs=[
            pl.BlockSpec((nb, C, S), lambda n: (n, 0, 0)),
            pl.BlockSpec((nb, C, S), lambda n: (n, 0, 0)),
            pl.BlockSpec((N, C, 2), lambda n: (0, 0, 0)),
            pl.BlockSpec((C, 1), lambda n: (0, 0)),
            pl.BlockSpec((C, 1), lambda n: (0, 0)),
            pl.BlockSpec((3 * 2 * C, 9 * 2 * C), lambda n: (0, 0)),
            pl.BlockSpec((3 * C, 9 * C), lambda n: (0, 0)),
            pl.BlockSpec((9, 1, S), lambda n: (0, 0, 0)),
            pl.BlockSpec((2, 1, S), lambda n: (0, 0, 0)),
        ],
        out_specs=(
            pl.BlockSpec((nb, C, S), lambda n: (n, 0, 0)),
            pl.BlockSpec((nb, C, 2), lambda n: (n, 0, 0)),
        ),
        scratch_shapes=[pltpu.VMEM((9 * 2 * C, S), _BF16),
                        pltpu.VMEM((9 * C, S), _BF16)],
        compiler_params=pltpu.CompilerParams(
            dimension_semantics=("parallel",)),
    )(xf, g1_pre, stats1, gam1, bet1, wdw, w2od, mhw, md)


def _stage3(y_pre, xf, stats2, gam2, bet2, eps):
    N, C, S = xf.shape
    nb = _pick_nb(N, 4)
    fn = functools.partial(_s3_kernel, nb=nb, count=float(N * S), eps=eps)
    return pl.pallas_call(
        fn,
        out_shape=jax.ShapeDtypeStruct((N, C, S), _F32),
        grid=(N // nb,),
        in_specs=[
            pl.BlockSpec((nb, C, S), lambda n: (n, 0, 0)),
            pl.BlockSpec((nb, C, S), lambda n: (n, 0, 0)),
            pl.BlockSpec((N, C, 2), lambda n: (0, 0, 0)),
            pl.BlockSpec((C, 1), lambda n: (0, 0)),
            pl.BlockSpec((C, 1), lambda n: (0, 0)),
        ],
        out_specs=pl.BlockSpec((nb, C, S), lambda n: (n, 0, 0)),
        compiler_params=pltpu.CompilerParams(
            dimension_semantics=("parallel",)),
    )(y_pre, xf, stats2, gam2, bet2)


def _tap_tables(D, H, W):
    S = D * H * W
    hw_deltas = tuple(oh * W + ow for oh in (-1, 0, 1) for ow in (-1, 0, 1))
    h = np.arange(H)[:, None]
    w = np.arange(W)[None, :]
    mhw = np.empty((9, 1, S), np.float32)
    for j, (oh, ow) in enumerate((a, b) for a in (-1, 0, 1)
                                 for b in (-1, 0, 1)):
        valid = ((0 <= h + oh) & (h + oh < H) &
                 (0 <= w + ow) & (w + ow < W))
        mhw[j, 0, :] = np.tile(valid.reshape(H * W), D).astype(np.float32)
    d = np.repeat(np.arange(D), H * W)
    md = np.stack([(d - 1 >= 0).astype(np.float32),
                   (d + 1 < D).astype(np.float32)])[:, None, :]
    return hw_deltas, jnp.asarray(mhw, _BF16), jnp.asarray(md)


def _pack_od(w, cout):
    k, cin, _ = w.shape
    w = w.reshape(3, 9, cin, cout)
    return jnp.transpose(w, (0, 3, 1, 2)).reshape(3 * cout, 9 * cin) \
        .astype(_BF16)


def _pack_dw(wx, wg):
    c2 = wx.shape[1] * 2
    wd = jnp.concatenate([wx, wg], axis=1).reshape(3, 9, c2)
    eye = jnp.eye(c2, dtype=_F32)
    blk = wd[:, :, None, :] * eye[None, None]
    return jnp.transpose(blk, (0, 2, 1, 3)).reshape(3 * c2, 9 * c2) \
        .astype(_BF16)


def kernel(feature, guidance, conv1_w, conv_w, x_kernel_diff,
           guidance_kernel_diff, conv1_bn_gamma, conv1_bn_beta,
           bn_gamma, bn_beta, eps=1e-5):
    N, C, D, H, W = feature.shape
    G = guidance.shape[1]
    S = D * H * W
    hw_span = H * W
    hw_deltas, mhw, md = _tap_tables(D, H, W)

    xf = feature.reshape(N, C, S).astype(_F32)
    gf = guidance.reshape(N, G, S).astype(_F32)

    w1od = _pack_od(conv1_w, C)
    w2od = _pack_od(conv_w, C)
    wdw = _pack_dw(x_kernel_diff, guidance_kernel_diff)

    gam1 = conv1_bn_gamma[:, None].astype(_F32)
    bet1 = conv1_bn_beta[:, None].astype(_F32)
    gam2 = bn_gamma[:, None].astype(_F32)
    bet2 = bn_beta[:, None].astype(_F32)

    g1_pre, stats1 = _stage1(gf, w1od, mhw, md, hw_deltas, hw_span)
    out_pre, stats2 = _stage2(xf, g1_pre, stats1, gam1, bet1, wdw, w2od,
                              mhw, md, hw_deltas, hw_span, eps)
    out = _stage3(out_pre, xf, stats2, gam2, bet2, eps)
    return out.reshape(N, C, D, H, W)


# speedup vs baseline: 1.7567x; 1.0053x over previous
import functools

import numpy as np

import jax
import jax.numpy as jnp
from jax.experimental import pallas as pl
from jax.experimental.pallas import tpu as pltpu

_F32 = jnp.float32
_BF16 = jnp.bfloat16


def _rot(x, delta):
    size = x.shape[-1]
    if delta % size == 0:
        return x
    return pltpu.roll(x, (-delta) % size, x.ndim - 1)


def _rot_bf(x, delta):
    size = x.shape[-1]
    d = delta % size
    if d == 0:
        return x
    return jnp.concatenate([x[..., d:], x[..., :d]], axis=-1)


def _build_r(y, mhw_ref, r_ref, hw_deltas, row0=0):
    rows = y.shape[0]
    stride = r_ref.shape[0] // len(hw_deltas)
    for j, delta in enumerate(hw_deltas):
        r = _rot_bf(y, delta)
        if j != 4:
            r = r * mhw_ref[j]
        base = j * stride + row0
        r_ref[base:base + rows, :] = r


def _od_combine(z, md_ref, rows, hw_span):
    lo = _rot(z[0:rows], -hw_span) * md_ref[0]
    hi = _rot(z[2 * rows:3 * rows], hw_span) * md_ref[1]
    return z[rows:2 * rows] + lo + hi


def _moments(y, stats_ref, b):
    stats_ref[b, :, 0:1] = jnp.sum(y, axis=1, keepdims=True)
    stats_ref[b, :, 1:2] = jnp.sum(y * y, axis=1, keepdims=True)


def _affine(stats_ref, gam_ref, bet_ref, count, eps):
    s = jnp.sum(stats_ref[...], axis=0)
    mean = s[:, 0:1] / count
    var = jnp.maximum(s[:, 1:2] / count - mean * mean, 0.0)
    scale = gam_ref[...] * jax.lax.rsqrt(var + eps)
    shift = bet_ref[...] - mean * scale
    return scale, shift


def _s1_kernel(g_ref, w1_ref, mhw_ref, md_ref, g1_ref, stats_ref, r_ref,
               *, hw_deltas, hw_span, nb):
    c = w1_ref.shape[0] // 3
    for b in range(nb):
        g = g_ref[b].astype(_BF16)
        _build_r(g, mhw_ref, r_ref, hw_deltas)
        z = jnp.dot(w1_ref[...], r_ref[...], preferred_element_type=_F32)
        g1 = _od_combine(z, md_ref, c, hw_span)
        _moments(g1, stats_ref, b)
        g1_ref[b] = g1.astype(_BF16)


def _s2_kernel(x_ref, g1_ref, stats1_ref, gam1_ref, bet1_ref, wdw_ref,
               w2_ref, mhw_ref, md_ref, out_ref, stats_ref, xbf_ref,
               rxg_ref, rp_ref, *, hw_deltas, hw_span, nb, count, eps):
    scale, shift = _affine(stats1_ref, gam1_ref, bet1_ref, count, eps)
    scale = scale.astype(_BF16)
    shift = shift.astype(_BF16)
    for b in range(nb):
        x = x_ref[b].astype(_BF16)
        xbf_ref[b] = x
        c = x.shape[0]
        g1 = g1_ref[b] * scale + shift
        _build_r(x, mhw_ref, rxg_ref, hw_deltas)
        _build_r(g1, mhw_ref, rxg_ref, hw_deltas, row0=c)
        zdw = jnp.dot(wdw_ref[...], rxg_ref[...],
                      preferred_element_type=_F32)
        dw = _od_combine(zdw, md_ref, 2 * c, hw_span)
        g_diff = dw[c:]
        prod = (dw[:c] * (g_diff * g_diff)).astype(_BF16)
        _build_r(prod, mhw_ref, rp_ref, hw_deltas)
        z2 = jnp.dot(w2_ref[...], rp_ref[...], preferred_element_type=_F32)
        y = _od_combine(z2, md_ref, c, hw_span)
        _moments(y, stats_ref, b)
        out_ref[b] = y.astype(_BF16)


def _s3_kernel(y_ref, x_ref, stats2_ref, gam_ref, bet_ref, o_ref,
               *, nb, count, eps):
    scale, shift = _affine(stats2_ref, gam_ref, bet_ref, count, eps)
    for b in range(nb):
        y = y_ref[b].astype(_F32) * scale + shift
        o_ref[b] = jnp.maximum(y, 0.0) + x_ref[b].astype(_F32)


def _pick_nb(n, cap):
    for b in range(cap, 0, -1):
        if n % b == 0:
            return b
    return 1


def _stage1(gf, w1od, mhw, md, hw_deltas, hw_span):
    N, G, S = gf.shape
    C = w1od.shape[0] // 3
    nb = _pick_nb(N, 4)
    fn = functools.partial(_s1_kernel, hw_deltas=hw_deltas, hw_span=hw_span,
                           nb=nb)
    return pl.pallas_call(
        fn,
        out_shape=(jax.ShapeDtypeStruct((N, C, S), _BF16),
                   jax.ShapeDtypeStruct((N, C, 2), _F32)),
        grid=(N // nb,),
        in_specs=[
            pl.BlockSpec((nb, G, S), lambda n: (n, 0, 0)),
            pl.BlockSpec((3 * C, 9 * G), lambda n: (0, 0)),
            pl.BlockSpec((9, 1, S), lambda n: (0, 0, 0)),
            pl.BlockSpec((2, 1, S), lambda n: (0, 0, 0)),
        ],
        out_specs=(
            pl.BlockSpec((nb, C, S), lambda n: (n, 0, 0)),
            pl.BlockSpec((nb, C, 2), lambda n: (n, 0, 0)),
        ),
        scratch_shapes=[pltpu.VMEM((9 * G, S), _BF16)],
        compiler_params=pltpu.CompilerParams(
            dimension_semantics=("parallel",)),
    )(gf, w1od, mhw, md)


def _stage2(xf, g1_pre, stats1, gam1, bet1, wdw, w2od, mhw, md, hw_deltas,
            hw_span, eps):
    N, C, S = xf.shape
    nb = _pick_nb(N, 2)
    fn = functools.partial(_s2_kernel, hw_deltas=hw_deltas, hw_span=hw_span,
                           nb=nb, count=float(N * S), eps=eps)
    return pl.pallas_call(
        fn,
        out_shape=(jax.ShapeDtypeStruct((N, C, S), _BF16),
                   jax.ShapeDtypeStruct((N, C, 2), _F32),
                   jax.ShapeDtypeStruct((N, C, S), _BF16)),
        grid=(N // nb,),
        in_specs=[
            pl.BlockSpec((nb, C, S), lambda n: (n, 0, 0)),
            pl.BlockSpec((nb, C, S), lambda n: (n, 0, 0)),
            pl.BlockSpec((N, C, 2), lambda n: (0, 0, 0)),
            pl.BlockSpec((C, 1), lambda n: (0, 0)),
            pl.BlockSpec((C, 1), lambda n: (0, 0)),
            pl.BlockSpec((3 * 2 * C, 9 * 2 * C), lambda n: (0, 0)),
            pl.BlockSpec((3 * C, 9 * C), lambda n: (0, 0)),
            pl.BlockSpec((9, 1, S), lambda n: (0, 0, 0)),
            pl.BlockSpec((2, 1, S), lambda n: (0, 0, 0)),
        ],
        out_specs=(
            pl.BlockSpec((nb, C, S), lambda n: (n, 0, 0)),
            pl.BlockSpec((nb, C, 2), lambda n: (n, 0, 0)),
            pl.BlockSpec((nb, C, S), lambda n: (n, 0, 0)),
        ),
        scratch_shapes=[pltpu.VMEM((9 * 2 * C, S), _BF16),
                        pltpu.VMEM((9 * C, S), _BF16)],
        compiler_params=pltpu.CompilerParams(
            dimension_semantics=("parallel",)),
    )(xf, g1_pre, stats1, gam1, bet1, wdw, w2od, mhw, md)


def _stage3(y_pre, xf, stats2, gam2, bet2, eps):
    N, C, S = xf.shape
    nb = _pick_nb(N, 4)
    fn = functools.partial(_s3_kernel, nb=nb, count=float(N * S), eps=eps)
    return pl.pallas_call(
        fn,
        out_shape=jax.ShapeDtypeStruct((N, C, S), _F32),
        grid=(N // nb,),
        in_specs=[
            pl.BlockSpec((nb, C, S), lambda n: (n, 0, 0)),
            pl.BlockSpec((nb, C, S), lambda n: (n, 0, 0)),
            pl.BlockSpec((N, C, 2), lambda n: (0, 0, 0)),
            pl.BlockSpec((C, 1), lambda n: (0, 0)),
            pl.BlockSpec((C, 1), lambda n: (0, 0)),
        ],
        out_specs=pl.BlockSpec((nb, C, S), lambda n: (n, 0, 0)),
        compiler_params=pltpu.CompilerParams(
            dimension_semantics=("parallel",)),
    )(y_pre, xf, stats2, gam2, bet2)


def _tap_tables(D, H, W):
    S = D * H * W
    hw_deltas = tuple(oh * W + ow for oh in (-1, 0, 1) for ow in (-1, 0, 1))
    h = np.arange(H)[:, None]
    w = np.arange(W)[None, :]
    mhw = np.empty((9, 1, S), np.float32)
    for j, (oh, ow) in enumerate((a, b) for a in (-1, 0, 1)
                                 for b in (-1, 0, 1)):
        valid = ((0 <= h + oh) & (h + oh < H) &
                 (0 <= w + ow) & (w + ow < W))
        mhw[j, 0, :] = np.tile(valid.reshape(H * W), D).astype(np.float32)
    d = np.repeat(np.arange(D), H * W)
    md = np.stack([(d - 1 >= 0).astype(np.float32),
                   (d + 1 < D).astype(np.float32)])[:, None, :]
    return hw_deltas, jnp.asarray(mhw, _BF16), jnp.asarray(md)


def _pack_od(w, cout):
    k, cin, _ = w.shape
    w = w.reshape(3, 9, cin, cout)
    return jnp.transpose(w, (0, 3, 1, 2)).reshape(3 * cout, 9 * cin) \
        .astype(_BF16)


def _pack_dw(wx, wg):
    c2 = wx.shape[1] * 2
    wd = jnp.concatenate([wx, wg], axis=1).reshape(3, 9, c2)
    eye = jnp.eye(c2, dtype=_F32)
    blk = wd[:, :, None, :] * eye[None, None]
    return jnp.transpose(blk, (0, 2, 1, 3)).reshape(3 * c2, 9 * c2) \
        .astype(_BF16)


def kernel(feature, guidance, conv1_w, conv_w, x_kernel_diff,
           guidance_kernel_diff, conv1_bn_gamma, conv1_bn_beta,
           bn_gamma, bn_beta, eps=1e-5):
    N, C, D, H, W = feature.shape
    G = guidance.shape[1]
    S = D * H * W
    hw_span = H * W
    hw_deltas, mhw, md = _tap_tables(D, H, W)

    xf = feature.reshape(N, C, S).astype(_F32)
    gf = guidance.reshape(N, G, S).astype(_F32)

    w1od = _pack_od(conv1_w, C)
    w2od = _pack_od(conv_w, C)
    wdw = _pack_dw(x_kernel_diff, guidance_kernel_diff)

    gam1 = conv1_bn_gamma[:, None].astype(_F32)
    bet1 = conv1_bn_beta[:, None].astype(_F32)
    gam2 = bn_gamma[:, None].astype(_F32)
    bet2 = bn_beta[:, None].astype(_F32)

    g1_pre, stats1 = _stage1(gf, w1od, mhw, md, hw_deltas, hw_span)
    out_pre, stats2, xbf = _stage2(xf, g1_pre, stats1, gam1, bet1, wdw, w2od,
                                   mhw, md, hw_deltas, hw_span, eps)
    out = _stage3(out_pre, xbf, stats2, gam2, bet2, eps)
    return out.reshape(N, C, D, H, W)
